# Initial kernel scaffold; baseline (speedup 1.0000x reference)
import jax
import jax.numpy as jnp
from jax.experimental import pallas as pl


def kernel(x, edge_index, eps, W1, b1, W2, b2, gamma, beta):
    raise NotImplementedError("write your pallas kernel here")


# speedup vs baseline: 4.8821x; 4.8821x over previous
import functools

import jax
import jax.numpy as jnp
from jax import lax
from jax.experimental import pallas as pl
from jax.experimental.pallas import tpu as pltpu
from jax.experimental.pallas import tpu_sc as plsc

_N = 10000
_D = 128
_E = 320000

_NC = 2
_NS = 16
_NW = _NC * _NS
_EPW = _E // _NW
_CHUNK = 80
_NCHUNK = _EPW // _CHUNK
_RPT = 624
_TAIL = _N - _NS * _RPT


def _sc_agg_body(x_hbm, ei_hbm, ej_hbm, zero_hbm, out_hbm,
                 acc, idx_i, idx_j, rows, sem):
    c = lax.axis_index("c")
    s = lax.axis_index("s")
    wid = s * _NC + c
    base = wid * _EPW
    r0 = s * _RPT

    pltpu.sync_copy(zero_hbm.at[pl.ds(r0, _RPT)], acc.at[pl.ds(r0, _RPT)])

    @pl.when(s == _NS - 1)
    def _():
        pltpu.sync_copy(zero_hbm.at[pl.ds(_NS * _RPT, _TAIL)],
                        acc.at[pl.ds(_NS * _RPT, _TAIL)])

    plsc.subcore_barrier()

    def chunk(t, carry):
        e0 = pl.multiple_of(base + t * _CHUNK, 8)
        pltpu.sync_copy(ej_hbm.at[pl.ds(e0, _CHUNK)], idx_j)
        pltpu.sync_copy(ei_hbm.at[pl.ds(e0, _CHUNK)], idx_i)
        pltpu.async_copy(x_hbm.at[idx_j], rows, sem).wait()
        pltpu.sync_copy(rows, acc.at[idx_i], add=True)
        return carry

    lax.fori_loop(0, _NCHUNK, chunk, 0)
    plsc.subcore_barrier()
    pltpu.sync_copy(acc.at[pl.ds(r0, _RPT)], out_hbm.at[c, pl.ds(r0, _RPT)])

    @pl.when(s == _NS - 1)
    def _():
        pltpu.sync_copy(acc.at[pl.ds(_NS * _RPT, _TAIL)],
                        out_hbm.at[c, pl.ds(_NS * _RPT, _TAIL)])


_sc_aggregate = functools.partial(
    pl.kernel,
    mesh=plsc.VectorSubcoreMesh(core_axis_name="c", subcore_axis_name="s"),
    out_type=jax.ShapeDtypeStruct((_NC, _N, _D), jnp.float32),
    scratch_types=[
        pltpu.VMEM_SHARED((_N, _D), jnp.float32),
        pltpu.VMEM((_CHUNK,), jnp.int32),
        pltpu.VMEM((_CHUNK,), jnp.int32),
        pltpu.VMEM((_CHUNK, _D), jnp.float32),
        pltpu.SemaphoreType.DMA,
    ],
)(_sc_agg_body)


_BLK = 1000


def _tc_mlp_body(scale_ref, x_ref, a0_ref, a1_ref, w1_ref, b1_ref,
                 w2_ref, b2_ref, g_ref, bt_ref, o_ref):
    h = scale_ref[0, 0] * x_ref[...] + a0_ref[...] + a1_ref[...]
    h = lax.dot_general(h, w1_ref[...], (((1,), (1,)), ((), ())),
                        preferred_element_type=jnp.float32) + b1_ref[...]
    h = jnp.maximum(h, 0.0)
    h = lax.dot_general(h, w2_ref[...], (((1,), (1,)), ((), ())),
                        preferred_element_type=jnp.float32) + b2_ref[...]
    mean = jnp.mean(h, axis=1, keepdims=True)
    d = h - mean
    var = jnp.mean(d * d, axis=1, keepdims=True)
    o_ref[...] = d * lax.rsqrt(var + 1e-5) * g_ref[...] + bt_ref[...]


def _tc_mlp(scale, x, a0, a1, w1, b1, w2, b2, gamma, beta):
    row_spec = pl.BlockSpec((_BLK, _D), lambda i: (i, 0))
    full = pl.BlockSpec((_D, _D), lambda i: (0, 0))
    vec = pl.BlockSpec((1, _D), lambda i: (0, 0))
    return pl.pallas_call(
        _tc_mlp_body,
        grid=(_N // _BLK,),
        in_specs=[
            pl.BlockSpec(memory_space=pltpu.SMEM),
            row_spec, row_spec, row_spec,
            full, vec, full, vec, vec, vec,
        ],
        out_specs=row_spec,
        out_shape=jax.ShapeDtypeStruct((_N, _D), jnp.float32),
    )(scale, x, a0, a1, w1, b1, w2, b2, gamma, beta)


def kernel(x, edge_index, eps, W1, b1, W2, b2, gamma, beta):
    ei = edge_index[0]
    ej = edge_index[1]
    zeros = jnp.zeros((_N, _D), jnp.float32)
    agg = _sc_aggregate(x, ei, ej, zeros)
    scale = (1.0 + eps).reshape(1, 1).astype(jnp.float32)
    return _tc_mlp(scale, x, agg[0], agg[1], W1,
                   b1.reshape(1, _D), W2, b2.reshape(1, _D),
                   gamma.reshape(1, _D), beta.reshape(1, _D))
